# baseline (speedup 1.0000x reference)
import jax
import jax.numpy as jnp
from jax.experimental import pallas as pl
from jax.experimental.pallas import tpu as pltpu

_N = 4096
_T = 512
_B = _N // _T
_W = 2 * _T
_NP = _N + _T
_THR = 0.5
_FAR = -1e9


def _nms_body(boxes_ref, boxes_t_ref, probs_ref, out_ref, keep_row_ref):
    riota = jax.lax.broadcasted_iota(jnp.int32, (_T, _T), 0)
    ciota = jax.lax.broadcasted_iota(jnp.int32, (_T, _T), 1)
    eye = riota == ciota
    tri = riota < ciota

    keep_row_ref[...] = jnp.ones((1, _NP), jnp.float32)

    def col_comps(base):
        x1c = boxes_ref[pl.ds(base, _T), 0:1]
        y1c = boxes_ref[pl.ds(base, _T), 1:2]
        x2c = boxes_ref[pl.ds(base, _T), 2:3]
        y2c = boxes_ref[pl.ds(base, _T), 3:4]
        return x1c, y1c, x2c, y2c

    def row_comps(base):
        x1r = boxes_t_ref[0:1, pl.ds(base, _T)]
        y1r = boxes_t_ref[1:2, pl.ds(base, _T)]
        x2r = boxes_t_ref[2:3, pl.ds(base, _T)]
        y2r = boxes_t_ref[3:4, pl.ds(base, _T)]
        return x1r, y1r, x2r, y2r

    def block_step(bi, carry):
        base_i = bi * _T
        x1c, y1c, x2c, y2c = col_comps(base_i)
        x1r, y1r, x2r, y2r = row_comps(base_i)

        area_c = (x2c - x1c) * (y2c - y1c)
        area_r = (x2r - x1r) * (y2r - y1r)
        w = jnp.maximum(jnp.minimum(x2c, x2r) - jnp.maximum(x1c, x1r), 0.0)
        h = jnp.maximum(jnp.minimum(y2c, y2r) - jnp.maximum(y1c, y1r), 0.0)
        inter = w * h
        union = area_c + area_r - inter
        diag_hit = (inter / jnp.maximum(union, 1e-9) > _THR) & tri

        init = keep_row_ref[0:1, pl.ds(base_i, _T)]

        def fix_body(c):
            kp, _ = c
            kcol = jnp.max(jnp.where(eye, kp, 0.0), axis=1, keepdims=True)
            supd = jnp.max(
                jnp.where((kcol > 0.0) & diag_hit, 1.0, 0.0),
                axis=0, keepdims=True)
            new = jnp.where(supd > 0.0, 0.0, init)
            return new, jnp.any(new != kp)

        keep_i, _ = jax.lax.while_loop(
            lambda c: c[1], fix_body, (init, jnp.bool_(True)))
        keep_row_ref[0:1, pl.ds(base_i, _T)] = keep_i

        kmask = jnp.max(jnp.where(eye, keep_i, 0.0), axis=1, keepdims=True) > 0.0
        mx1 = jnp.where(kmask, x1c, _FAR)
        my1 = jnp.where(kmask, y1c, _FAR)
        mx2 = jnp.where(kmask, x2c, _FAR)
        my2 = jnp.where(kmask, y2c, _FAR)
        mac = (mx2 - mx1) * (my2 - my1)
        X1 = jnp.broadcast_to(mx1, (_T, _W))
        Y1 = jnp.broadcast_to(my1, (_T, _W))
        X2 = jnp.broadcast_to(mx2, (_T, _W))
        Y2 = jnp.broadcast_to(my2, (_T, _W))
        AC = jnp.broadcast_to(mac, (_T, _W))

        def sweep(s, c):
            base_j = (bi + 1) * _T + s * _W
            x1 = boxes_t_ref[0:1, pl.ds(base_j, _W)]
            y1 = boxes_t_ref[1:2, pl.ds(base_j, _W)]
            x2 = boxes_t_ref[2:3, pl.ds(base_j, _W)]
            y2 = boxes_t_ref[3:4, pl.ds(base_j, _W)]
            ar = (x2 - x1) * (y2 - y1)
            ww = jnp.maximum(jnp.minimum(X2, x2) - jnp.maximum(X1, x1), 0.0)
            hh = jnp.maximum(jnp.minimum(Y2, y2) - jnp.maximum(Y1, y1), 0.0)
            it = ww * hh
            un = AC + ar - it
            iou = it / jnp.maximum(un, 1e-9)
            hit = jnp.max(iou, axis=0, keepdims=True)
            kb = keep_row_ref[0:1, pl.ds(base_j, _W)]
            keep_row_ref[0:1, pl.ds(base_j, _W)] = jnp.where(
                hit > _THR, 0.0, kb)
            return c

        jax.lax.fori_loop(0, (_B - bi) // 2, sweep, 0)
        return carry

    jax.lax.fori_loop(0, _B, block_step, 0)

    kr = keep_row_ref[0:1, 0:_N]
    out_ref[0:1, :] = boxes_t_ref[0:1, 0:_N] * kr
    out_ref[1:2, :] = boxes_t_ref[1:2, 0:_N] * kr
    out_ref[2:3, :] = boxes_t_ref[2:3, 0:_N] * kr
    out_ref[3:4, :] = boxes_t_ref[3:4, 0:_N] * kr
    out_ref[4:5, :] = probs_ref[0:1, :] * kr


def _nms_call(top_boxes, boxes_t, probs_row, interpret=False):
    return pl.pallas_call(
        _nms_body,
        out_shape=jax.ShapeDtypeStruct((5, _N), jnp.float32),
        scratch_shapes=[
            pltpu.VMEM((1, _NP), jnp.float32),
        ],
        interpret=interpret,
    )(top_boxes, boxes_t, probs_row)


def kernel(boxes, scores):
    probs = jax.nn.sigmoid(scores)
    top_probs, top_idx = jax.lax.top_k(probs, _N)
    top_boxes = boxes[top_idx]
    boxes_t_pad = jnp.concatenate(
        [top_boxes.T, jnp.full((4, _T), _FAR, jnp.float32)], axis=1)
    out_t = _nms_call(top_boxes, boxes_t_pad, top_probs[None, :])
    return out_t.T


# speedup vs baseline: 1.0375x; 1.0375x over previous
import jax
import jax.numpy as jnp
from jax.experimental import pallas as pl
from jax.experimental.pallas import tpu as pltpu

_N = 4096
_T = 512
_B = _N // _T
_W = 2 * _T
_NP = _N + _T
_THR = 0.5
_FAR = -1e9


def _nms_body(boxes_ref, boxes_t_ref, probs_ref, out_ref, keep_row_ref):
    riota = jax.lax.broadcasted_iota(jnp.int32, (_T, _T), 0)
    ciota = jax.lax.broadcasted_iota(jnp.int32, (_T, _T), 1)
    eye = riota == ciota
    tri = riota < ciota

    keep_row_ref[...] = jnp.ones((1, _NP), jnp.float32)

    def col_comps(base):
        x1c = boxes_ref[pl.ds(base, _T), 0:1]
        y1c = boxes_ref[pl.ds(base, _T), 1:2]
        x2c = boxes_ref[pl.ds(base, _T), 2:3]
        y2c = boxes_ref[pl.ds(base, _T), 3:4]
        return x1c, y1c, x2c, y2c

    def row_comps(base):
        x1r = boxes_t_ref[0:1, pl.ds(base, _T)]
        y1r = boxes_t_ref[1:2, pl.ds(base, _T)]
        x2r = boxes_t_ref[2:3, pl.ds(base, _T)]
        y2r = boxes_t_ref[3:4, pl.ds(base, _T)]
        return x1r, y1r, x2r, y2r

    def block_step(bi, carry):
        base_i = bi * _T
        x1c, y1c, x2c, y2c = col_comps(base_i)
        x1r, y1r, x2r, y2r = row_comps(base_i)

        area_c = (x2c - x1c) * (y2c - y1c)
        area_r = (x2r - x1r) * (y2r - y1r)
        w = jnp.maximum(jnp.minimum(x2c, x2r) - jnp.maximum(x1c, x1r), 0.0)
        h = jnp.maximum(jnp.minimum(y2c, y2r) - jnp.maximum(y1c, y1r), 0.0)
        inter = w * h
        union = area_c + area_r - inter
        diag_hit = (inter / jnp.maximum(union, 1e-9) > _THR) & tri

        init = keep_row_ref[0:1, pl.ds(base_i, _T)]

        def fix_step(kp):
            kcol = jnp.max(jnp.where(eye, kp, 0.0), axis=1, keepdims=True)
            supd = jnp.max(
                jnp.where((kcol > 0.0) & diag_hit, 1.0, 0.0),
                axis=0, keepdims=True)
            return jnp.where(supd > 0.0, 0.0, init)

        def fix_body(c):
            kp, _ = c
            new = fix_step(fix_step(kp))
            return new, jnp.any(new != kp)

        keep_i, _ = jax.lax.while_loop(
            lambda c: c[1], fix_body, (init, jnp.bool_(True)))
        keep_row_ref[0:1, pl.ds(base_i, _T)] = keep_i

        kmask = jnp.max(jnp.where(eye, keep_i, 0.0), axis=1, keepdims=True) > 0.0
        mx1 = jnp.where(kmask, x1c, _FAR)
        my1 = jnp.where(kmask, y1c, _FAR)
        mx2 = jnp.where(kmask, x2c, _FAR)
        my2 = jnp.where(kmask, y2c, _FAR)
        mac = (mx2 - mx1) * (my2 - my1)
        X1 = jnp.broadcast_to(mx1, (_T, _T))
        Y1 = jnp.broadcast_to(my1, (_T, _T))
        X2 = jnp.broadcast_to(mx2, (_T, _T))
        Y2 = jnp.broadcast_to(my2, (_T, _T))
        AC = jnp.broadcast_to(mac, (_T, _T))

        def sweep(bj, c):
            base_j = bj * _T
            x1, y1, x2, y2 = row_comps(base_j)
            ar = (x2 - x1) * (y2 - y1)
            ww = jnp.maximum(jnp.minimum(X2, x2) - jnp.maximum(X1, x1), 0.0)
            hh = jnp.maximum(jnp.minimum(Y2, y2) - jnp.maximum(Y1, y1), 0.0)
            it = ww * hh
            un = AC + ar - it
            iou = it / jnp.maximum(un, 1e-9)
            hit = jnp.max(iou, axis=0, keepdims=True)
            kb = keep_row_ref[0:1, pl.ds(base_j, _T)]
            keep_row_ref[0:1, pl.ds(base_j, _T)] = jnp.where(
                hit > _THR, 0.0, kb)
            return c

        jax.lax.fori_loop(bi + 1, _B, sweep, 0)
        return carry

    jax.lax.fori_loop(0, _B, block_step, 0)

    kr = keep_row_ref[0:1, 0:_N]
    out_ref[0:1, :] = boxes_t_ref[0:1, 0:_N] * kr
    out_ref[1:2, :] = boxes_t_ref[1:2, 0:_N] * kr
    out_ref[2:3, :] = boxes_t_ref[2:3, 0:_N] * kr
    out_ref[3:4, :] = boxes_t_ref[3:4, 0:_N] * kr
    out_ref[4:5, :] = probs_ref[0:1, :] * kr


def _nms_call(top_boxes, boxes_t, probs_row, interpret=False):
    return pl.pallas_call(
        _nms_body,
        out_shape=jax.ShapeDtypeStruct((5, _N), jnp.float32),
        scratch_shapes=[
            pltpu.VMEM((1, _NP), jnp.float32),
        ],
        interpret=interpret,
    )(top_boxes, boxes_t, probs_row)


def kernel(boxes, scores):
    probs = jax.nn.sigmoid(scores)
    top_probs, top_idx = jax.lax.top_k(probs, _N)
    top_boxes = boxes[top_idx]
    boxes_t_pad = jnp.concatenate(
        [top_boxes.T, jnp.full((4, _T), _FAR, jnp.float32)], axis=1)
    out_t = _nms_call(top_boxes, boxes_t_pad, top_probs[None, :])
    return out_t.T


# speedup vs baseline: 1.0644x; 1.0260x over previous
import jax
import jax.numpy as jnp
from jax.experimental import pallas as pl
from jax.experimental.pallas import tpu as pltpu

_N = 4096
_T = 512
_B = _N // _T
_THR = 0.5
_FAR = -1e9


def _nms_body(boxes_ref, boxes_t_ref, probs_ref, out_ref, keep_row_ref):
    riota = jax.lax.broadcasted_iota(jnp.int32, (_T, _T), 0)
    ciota = jax.lax.broadcasted_iota(jnp.int32, (_T, _T), 1)
    eye = riota == ciota
    tri = riota < ciota

    keep_row_ref[...] = jnp.ones((1, _N), jnp.float32)

    def col_comps(base):
        x1c = boxes_ref[pl.ds(base, _T), 0:1]
        y1c = boxes_ref[pl.ds(base, _T), 1:2]
        x2c = boxes_ref[pl.ds(base, _T), 2:3]
        y2c = boxes_ref[pl.ds(base, _T), 3:4]
        return x1c, y1c, x2c, y2c

    def row_comps(base):
        x1r = boxes_t_ref[0:1, pl.ds(base, _T)]
        y1r = boxes_t_ref[1:2, pl.ds(base, _T)]
        x2r = boxes_t_ref[2:3, pl.ds(base, _T)]
        y2r = boxes_t_ref[3:4, pl.ds(base, _T)]
        return x1r, y1r, x2r, y2r

    def block_step(bi, carry):
        base_i = bi * _T
        x1c, y1c, x2c, y2c = col_comps(base_i)
        x1r, y1r, x2r, y2r = row_comps(base_i)

        area_c = (x2c - x1c) * (y2c - y1c)
        area_r = (x2r - x1r) * (y2r - y1r)
        w = jnp.maximum(jnp.minimum(x2c, x2r) - jnp.maximum(x1c, x1r), 0.0)
        h = jnp.maximum(jnp.minimum(y2c, y2r) - jnp.maximum(y1c, y1r), 0.0)
        inter = w * h
        union = area_c + area_r - inter
        diag_hit = (inter / jnp.maximum(union, 1e-9) > _THR) & tri

        init = keep_row_ref[0:1, pl.ds(base_i, _T)]

        def fix_body(c):
            kp, _ = c
            kcol = jnp.max(jnp.where(eye, kp, 0.0), axis=1, keepdims=True)
            supd = jnp.max(
                jnp.where((kcol > 0.0) & diag_hit, 1.0, 0.0),
                axis=0, keepdims=True)
            new = jnp.where(supd > 0.0, 0.0, init)
            return new, jnp.any(new != kp)

        keep_i, _ = jax.lax.while_loop(
            lambda c: c[1], fix_body, (init, jnp.bool_(True)))
        keep_row_ref[0:1, pl.ds(base_i, _T)] = keep_i

        kmask = jnp.max(jnp.where(eye, keep_i, 0.0), axis=1, keepdims=True) > 0.0
        mx1 = jnp.where(kmask, x1c, _FAR)
        my1 = jnp.where(kmask, y1c, _FAR)
        mx2 = jnp.where(kmask, x2c, _FAR)
        my2 = jnp.where(kmask, y2c, _FAR)
        mac = (mx2 - mx1) * (my2 - my1)
        X1 = jnp.broadcast_to(mx1, (_T, _T))
        Y1 = jnp.broadcast_to(my1, (_T, _T))
        X2 = jnp.broadcast_to(mx2, (_T, _T))
        Y2 = jnp.broadcast_to(my2, (_T, _T))
        AC = jnp.broadcast_to(mac, (_T, _T))

        def sweep(bj, c):
            base_j = bj * _T
            x1, y1, x2, y2 = row_comps(base_j)
            ar = (x2 - x1) * (y2 - y1)
            ww = jnp.maximum(jnp.minimum(X2, x2) - jnp.maximum(X1, x1), 0.0)
            hh = jnp.maximum(jnp.minimum(Y2, y2) - jnp.maximum(Y1, y1), 0.0)
            it = ww * hh
            un = AC + ar - it
            iou = it / jnp.maximum(un, 1e-9)
            hit = jnp.max(iou, axis=0, keepdims=True)
            kb = keep_row_ref[0:1, pl.ds(base_j, _T)]
            keep_row_ref[0:1, pl.ds(base_j, _T)] = jnp.where(
                hit > _THR, 0.0, kb)
            return c

        jax.lax.fori_loop(bi + 1, _B, sweep, 0)
        return carry

    jax.lax.fori_loop(0, _B, block_step, 0)

    kr = keep_row_ref[0:1, :]
    out_ref[0:1, :] = boxes_t_ref[0:1, :] * kr
    out_ref[1:2, :] = boxes_t_ref[1:2, :] * kr
    out_ref[2:3, :] = boxes_t_ref[2:3, :] * kr
    out_ref[3:4, :] = boxes_t_ref[3:4, :] * kr
    out_ref[4:5, :] = probs_ref[0:1, :] * kr


def _nms_call(top_boxes, boxes_t, probs_row, interpret=False):
    return pl.pallas_call(
        _nms_body,
        out_shape=jax.ShapeDtypeStruct((5, _N), jnp.float32),
        scratch_shapes=[
            pltpu.VMEM((1, _N), jnp.float32),
        ],
        interpret=interpret,
    )(top_boxes, boxes_t, probs_row)


def kernel(boxes, scores):
    probs = jax.nn.sigmoid(scores)
    top_probs, top_idx = jax.lax.top_k(probs, _N)
    top_boxes = boxes[top_idx]
    out_t = _nms_call(top_boxes, top_boxes.T, top_probs[None, :])
    return out_t.T
